# baseline (speedup 1.0000x reference)
import jax
import jax.numpy as jnp
from jax.experimental import pallas as pl
from jax.experimental.pallas import tpu as pltpu

_N = 10000
_NP = 10240
_BLK = 1280
_NB = _NP // _BLK
_F = 128
_MININT = -(2 ** 31)
_KS = (5000, 2500, 1250)


def _f32_key(s):
    i = jax.lax.bitcast_convert_type(s, jnp.int32)
    return jnp.where(i >= 0, i, i ^ jnp.int32(0x7FFFFFFF))


def _kth_largest_thresh(keys8, k):
    d = jax.lax.broadcasted_iota(jnp.int32, (8, 1), 0)
    t_u = jnp.int32(0)
    for shift in (29, 26, 23, 20, 17, 14, 11, 8, 5, 2):
        cand = t_u | (d << shift)
        thr = cand ^ jnp.int32(_MININT)
        cnt = jnp.sum((keys8 >= thr).astype(jnp.int32), axis=1, keepdims=True)
        dstar = jnp.sum(((cnt >= k) & (d >= 1)).astype(jnp.int32))
        t_u = t_u | (dstar << shift)
    cand = t_u | d
    thr = cand ^ jnp.int32(_MININT)
    cnt = jnp.sum((keys8 >= thr).astype(jnp.int32), axis=1, keepdims=True)
    dstar = jnp.sum(((cnt >= k) & (d >= 1) & (d <= 3)).astype(jnp.int32))
    return (t_u | dstar) ^ jnp.int32(_MININT)


def _fused_kernel(x_ref, w1_ref, b1_ref, w2_ref, b2_ref, p_ref,
                  n_ref, out_ref, xw_s):
    i = pl.program_id(0)
    h = jnp.maximum(
        jnp.dot(x_ref[...], w1_ref[...], preferred_element_type=jnp.float32)
        + b1_ref[...], 0.0)
    x0 = jnp.dot(h, w2_ref[...], preferred_element_type=jnp.float32) + b2_ref[...]
    row = jax.lax.broadcasted_iota(jnp.int32, (_BLK, 1), 0) + i * _BLK
    xw_s[pl.ds(i * _BLK, _BLK), :] = jnp.where(row < _N, x0, 0.0)

    @pl.when(i == _NB - 1)
    def _select_and_reduce():
        alive_l = jax.lax.broadcasted_iota(jnp.int32, (1, _NP), 1) < _N
        for r, k in enumerate(_KS):
            xw = xw_s[...]
            s_l = jax.lax.dot_general(
                p_ref[...], xw, (((1,), (1,)), ((), ())),
                preferred_element_type=jnp.float32)[r:r + 1, :] / n_ref[0:1, r:r + 1]
            keys = jnp.where(alive_l, _f32_key(s_l), jnp.int32(_MININT))
            t = _kth_largest_thresh(jnp.broadcast_to(keys, (8, _NP)), k)
            alive_l = keys >= t
            t_l = jnp.where(alive_l, jnp.tanh(s_l), 0.0)
            if r < 2:
                xw_s[...] = xw * jnp.reshape(t_l, (_NP, 1))
            else:
                prod = xw * jnp.reshape(t_l, (_NP, 1))
                out_ref[...] = jnp.sum(prod, axis=0, keepdims=True) * (1.0 / _KS[-1])


def kernel(x, edge_index, edge_attr, batch, params):
    del edge_index, edge_attr, batch
    nm = params['node_mlp']
    pool = params['pool']
    p = jnp.stack(pool, axis=0)
    p = jnp.pad(p, ((0, 8 - len(pool)), (0, 0)))
    norms = jnp.stack([jnp.linalg.norm(pv) for pv in pool])
    norms = jnp.pad(norms, (0, 8 - len(pool)), constant_values=1.0)[None, :]
    full = lambda shape: pl.BlockSpec(shape, lambda i: (0, 0))
    out = pl.pallas_call(
        _fused_kernel,
        grid=(_NB,),
        in_specs=[
            pl.BlockSpec((_BLK, _F), lambda i: (i, 0)),
            full((_F, _F)), full((1, _F)), full((_F, _F)), full((1, _F)),
            full((8, _F)), full((1, 8)),
        ],
        out_specs=full((1, _F)),
        out_shape=jax.ShapeDtypeStruct((1, _F), jnp.float32),
        scratch_shapes=[
            pltpu.VMEM((_NP, _F), jnp.float32),
        ],
    )(x, nm['W1'], nm['b1'][None, :], nm['W2'], nm['b2'][None, :], p, norms)
    return out


# speedup vs baseline: 1.1309x; 1.1309x over previous
import jax
import jax.numpy as jnp
from jax.experimental import pallas as pl
from jax.experimental.pallas import tpu as pltpu

_N = 10000
_NP = 10240
_BLK = 5120
_NB = _NP // _BLK
_F = 128
_MININT = -(2 ** 31)
_KS = (5000, 2500, 1250)


def _f32_key(s):
    i = jax.lax.bitcast_convert_type(s, jnp.int32)
    return jnp.where(i >= 0, i, i ^ jnp.int32(0x7FFFFFFF))


def _kth_largest_thresh(keys8, k):
    d = jax.lax.broadcasted_iota(jnp.int32, (8, 1), 0)
    t_u = jnp.int32(0)
    for shift in (29, 26, 23, 20, 17, 14, 11, 8, 5, 2):
        cand = t_u | (d << shift)
        thr = cand ^ jnp.int32(_MININT)
        cnt = jnp.sum((keys8 >= thr).astype(jnp.int32), axis=1, keepdims=True)
        dstar = jnp.sum(((cnt >= k) & (d >= 1)).astype(jnp.int32))
        t_u = t_u | (dstar << shift)
    cand = t_u | d
    thr = cand ^ jnp.int32(_MININT)
    cnt = jnp.sum((keys8 >= thr).astype(jnp.int32), axis=1, keepdims=True)
    dstar = jnp.sum(((cnt >= k) & (d >= 1) & (d <= 3)).astype(jnp.int32))
    return (t_u | dstar) ^ jnp.int32(_MININT)


def _fused_kernel(x_ref, w1_ref, b1_ref, w2_ref, b2_ref, p_ref,
                  n_ref, out_ref, xw_s):
    i = pl.program_id(0)
    h = jnp.maximum(
        jnp.dot(x_ref[...], w1_ref[...], preferred_element_type=jnp.float32)
        + b1_ref[...], 0.0)
    x0 = jnp.dot(h, w2_ref[...], preferred_element_type=jnp.float32) + b2_ref[...]
    row = jax.lax.broadcasted_iota(jnp.int32, (_BLK, 1), 0) + i * _BLK
    xw_s[pl.ds(i * _BLK, _BLK), :] = jnp.where(row < _N, x0, 0.0)

    @pl.when(i == _NB - 1)
    def _select_and_reduce():
        alive_l = jax.lax.broadcasted_iota(jnp.int32, (1, _NP), 1) < _N
        for r, k in enumerate(_KS):
            xw = xw_s[...]
            s_l = jax.lax.dot_general(
                p_ref[...], xw, (((1,), (1,)), ((), ())),
                preferred_element_type=jnp.float32)[r:r + 1, :] / n_ref[0:1, r:r + 1]
            keys = jnp.where(alive_l, _f32_key(s_l), jnp.int32(_MININT))
            t = _kth_largest_thresh(jnp.broadcast_to(keys, (8, _NP)), k)
            alive_l = keys >= t
            t_l = jnp.where(alive_l, jnp.tanh(s_l), 0.0)
            if r < 2:
                xw_s[...] = xw * jnp.reshape(t_l, (_NP, 1))
            else:
                prod = xw * jnp.reshape(t_l, (_NP, 1))
                out_ref[...] = jnp.sum(prod, axis=0, keepdims=True) * (1.0 / _KS[-1])


def kernel(x, edge_index, edge_attr, batch, params):
    del edge_index, edge_attr, batch
    nm = params['node_mlp']
    pool = params['pool']
    p = jnp.stack(pool, axis=0)
    p = jnp.pad(p, ((0, 8 - len(pool)), (0, 0)))
    norms = jnp.stack([jnp.linalg.norm(pv) for pv in pool])
    norms = jnp.pad(norms, (0, 8 - len(pool)), constant_values=1.0)[None, :]
    full = lambda shape: pl.BlockSpec(shape, lambda i: (0, 0))
    out = pl.pallas_call(
        _fused_kernel,
        grid=(_NB,),
        in_specs=[
            pl.BlockSpec((_BLK, _F), lambda i: (i, 0)),
            full((_F, _F)), full((1, _F)), full((_F, _F)), full((1, _F)),
            full((8, _F)), full((1, 8)),
        ],
        out_specs=full((1, _F)),
        out_shape=jax.ShapeDtypeStruct((1, _F), jnp.float32),
        scratch_shapes=[
            pltpu.VMEM((_NP, _F), jnp.float32),
        ],
    )(x, nm['W1'], nm['b1'][None, :], nm['W2'], nm['b2'][None, :], p, norms)
    return out
